# baseline (speedup 1.0000x reference)
import jax
import jax.numpy as jnp
from jax import lax
from jax.experimental import pallas as pl
from jax.experimental.pallas import tpu as pltpu
from jax.experimental.pallas import tpu_sc as plsc

B = 4096
PAD = 50
PADP = 56
EMB = 32
NEG = -1e9

NC = 2
NS = 16
NW = NC * NS

M = B * PADP
MH_W = M // NW
CH = 1792
N_CH = MH_W // CH
SM_W = B // NW


def _sc_gather_body(
    t_hist_item, t_hist_cate, t_uid, t_uage, t_ugen, t_iid, t_icate,
    idx_hist_item, idx_hist_cate, idx_uid, idx_uage, idx_ugen, idx_iid,
    idx_icate,
    o_hist_item, o_hist_cate, o_uid, o_uage, o_ugen, o_iid, o_icate,
    idx2, buf2, idx_sml, buf_sml, ct_sh, gsems, wsems, ssem,
):
  sid = lax.axis_index("s")
  wid = lax.axis_index("c") * NS + sid
  sbase = wid * SM_W

  @pl.when(sid == 0)
  def _():
    pltpu.sync_copy(t_hist_cate, ct_sh)

  pltpu.sync_copy(idx_hist_item.at[wid], idx2.at[0])
  pltpu.sync_copy(idx_hist_cate.at[wid], idx2.at[1])
  plsc.subcore_barrier()

  smalls = ((t_uid, idx_uid, o_uid), (t_uage, idx_uage, o_uage),
            (t_ugen, idx_ugen, o_ugen), (t_iid, idx_iid, o_iid),
            (t_icate, idx_icate, o_icate))
  sm_g = []
  for i, (table, idx_hbm, _) in enumerate(smalls):
    pltpu.sync_copy(idx_hbm.at[pl.ds(sbase, SM_W)], idx_sml.at[i])
    sm_g.append(pltpu.async_copy(table.at[idx_sml.at[i]], buf_sml.at[i], ssem))

  tables = (t_hist_item, ct_sh)
  outs = (o_hist_item, o_hist_cate)
  units = [(t, ch) for t in range(2) for ch in range(N_CH)]
  fires = [None] * len(units)
  wbs = [None] * len(units)

  def fire(u):
    t, ch = units[u]
    b = u % 2
    fires[u] = pltpu.async_copy(
        tables[t].at[idx2.at[t].at[pl.ds(ch * CH, CH)]],
        buf2.at[b], gsems[b])

  def drain_and_writeback(u):
    t, ch = units[u]
    b = u % 2
    fires[u].wait()
    wbs[u] = pltpu.async_copy(
        buf2.at[b], outs[t].at[pl.ds(wid * MH_W + ch * CH, CH)], wsems[b])

  fire(0)
  for u in range(1, len(units)):
    if u >= 2:
      wbs[u - 2].wait()
    fire(u)
    drain_and_writeback(u - 1)
  drain_and_writeback(len(units) - 1)
  wbs[-2].wait()
  wbs[-1].wait()

  for c in sm_g:
    c.wait()
  for i, (_, _, out_hbm) in enumerate(smalls):
    pltpu.sync_copy(buf_sml.at[i], out_hbm.at[pl.ds(sbase, SM_W)])


def _cast_body(x_ref, o_ref):
  o_ref[...] = x_ref[...].astype(jnp.bfloat16)


def _bf16(x, rows_per_block):
  rows = x.shape[0]
  return pl.pallas_call(
      _cast_body,
      grid=(rows // rows_per_block,),
      in_specs=[pl.BlockSpec((rows_per_block, EMB), lambda i: (i, 0))],
      out_specs=pl.BlockSpec((rows_per_block, EMB), lambda i: (i, 0)),
      out_shape=jax.ShapeDtypeStruct((rows, EMB), jnp.bfloat16),
  )(x)


def _sc_gather(tables, idxs):
  f32 = jnp.float32
  bf16 = jnp.bfloat16
  out_type = [
      jax.ShapeDtypeStruct((M, EMB), bf16),
      jax.ShapeDtypeStruct((M, EMB), bf16),
      jax.ShapeDtypeStruct((B, EMB), f32),
      jax.ShapeDtypeStruct((B, EMB), f32),
      jax.ShapeDtypeStruct((B, EMB), f32),
      jax.ShapeDtypeStruct((B, EMB), f32),
      jax.ShapeDtypeStruct((B, EMB), f32),
  ]
  mesh = plsc.VectorSubcoreMesh(core_axis_name="c", subcore_axis_name="s",
                                num_cores=NC, num_subcores=NS)
  run = pl.kernel(
      _sc_gather_body,
      out_type=out_type,
      mesh=mesh,
      compiler_params=pltpu.CompilerParams(use_tc_tiling_on_sc=False,
                                           disable_bounds_checks=True),
      scratch_types=[
          pltpu.VMEM((2, MH_W), jnp.int32),
          pltpu.VMEM((2, CH, EMB), bf16),
          pltpu.VMEM((5, SM_W), jnp.int32),
          pltpu.VMEM((5, SM_W, EMB), f32),
          pltpu.VMEM_SHARED((1000, EMB), bf16),
          (pltpu.SemaphoreType.DMA, pltpu.SemaphoreType.DMA),
          (pltpu.SemaphoreType.DMA, pltpu.SemaphoreType.DMA),
          pltpu.SemaphoreType.DMA,
      ],
  )
  return run(*tables, *idxs)


BB = 128
MBP = BB * PADP


def _tc_body(seq_ref, ki_ref, kc_ref, qi_ref, qc_ref, uid_ref, uage_ref,
             ugen_ref, w0_ref, b0_ref, w1_ref, b1_ref, w2_ref, b2_ref,
             fw0_ref, fb0_ref, fw1_ref, fb1_ref, fw2_ref, fb2_ref,
             fw3_ref, fb3_ref, out_ref):
  f32 = jnp.float32
  qi = qi_ref[...]
  qc = qc_ref[...]
  q = jnp.concatenate([qi, qc], axis=-1)
  w0 = w0_ref[...]
  w0q, w0k, w0d, w0p = (w0[0:64], w0[64:128], w0[128:192], w0[192:256])
  a_row = jnp.dot(q, w0q + w0d, preferred_element_type=f32)

  ki = ki_ref[...].astype(f32).reshape(BB, PADP, EMB)
  kc = kc_ref[...].astype(f32).reshape(BB, PADP, EMB)
  k3 = jnp.concatenate([ki, kc], axis=-1)
  qk3 = q[:, None, :] * k3
  k2 = k3.reshape(MBP, 2 * EMB)
  x = jnp.concatenate([k2, qk3.reshape(MBP, 2 * EMB)], axis=-1)
  wx = jnp.concatenate([w0k - w0d, w0p], axis=0)
  h = jnp.dot(x, wx, preferred_element_type=f32)
  h += jnp.broadcast_to(a_row[:, None, :], (BB, PADP, 80)).reshape(MBP, 80)
  h = jnp.maximum(h + b0_ref[...], 0.0)
  h = jnp.maximum(jnp.dot(h, w1_ref[...], preferred_element_type=f32)
                  + b1_ref[...], 0.0)
  s = jnp.dot(h, w2_ref[...], preferred_element_type=f32) + b2_ref[...]
  s3 = s.reshape(BB, PADP, 1)
  pos = lax.broadcasted_iota(jnp.int32, (BB, PADP, 1), 1)
  mask = pos < seq_ref[...].reshape(BB, 1, 1)
  s3 = jnp.where(mask, s3, NEG)
  mx = jnp.max(s3, axis=1, keepdims=True)
  e = jnp.exp(s3 - mx)
  att = e / jnp.sum(e, axis=1, keepdims=True)
  seq_emb = jnp.sum(att * k3, axis=1)

  dnn_in = jnp.concatenate(
      [uid_ref[...], uage_ref[...], ugen_ref[...], seq_emb, qi, qc], axis=-1)
  h = jnp.maximum(jnp.dot(dnn_in, fw0_ref[...], preferred_element_type=f32)
                  + fb0_ref[...], 0.0)
  h = jnp.maximum(jnp.dot(h, fw1_ref[...], preferred_element_type=f32)
                  + fb1_ref[...], 0.0)
  h = jnp.maximum(jnp.dot(h, fw2_ref[...], preferred_element_type=f32)
                  + fb2_ref[...], 0.0)
  z = jnp.dot(h, fw3_ref[...], preferred_element_type=f32) + fb3_ref[...]
  out_ref[...] = 1.0 / (1.0 + jnp.exp(-z))


def _tc_dense(seq_len2, k_item, k_cate, q_item, q_cate, u_id, u_age, u_gen,
              att_w0, att_b0, att_w1, att_b1, att_w2, att_b2,
              f_w0, f_b0, f_w1, f_b1, f_w2, f_b2, f_w3, f_b3):
  grid = (B // BB,)
  bcast = lambda i: (0, 0)

  def row_spec(cols):
    return pl.BlockSpec((BB, cols), lambda i: (i, 0))

  in_specs = [
      pl.BlockSpec((BB, 1), lambda i: (i, 0)),
      pl.BlockSpec((MBP, EMB), lambda i: (i, 0)),
      pl.BlockSpec((MBP, EMB), lambda i: (i, 0)),
      row_spec(EMB), row_spec(EMB),
      row_spec(EMB), row_spec(EMB), row_spec(EMB),
      pl.BlockSpec((256, 80), bcast), pl.BlockSpec((1, 80), bcast),
      pl.BlockSpec((80, 40), bcast), pl.BlockSpec((1, 40), bcast),
      pl.BlockSpec((40, 1), bcast), pl.BlockSpec((1, 1), bcast),
      pl.BlockSpec((224, 256), bcast), pl.BlockSpec((1, 256), bcast),
      pl.BlockSpec((256, 128), bcast), pl.BlockSpec((1, 128), bcast),
      pl.BlockSpec((128, 64), bcast), pl.BlockSpec((1, 64), bcast),
      pl.BlockSpec((64, 1), bcast), pl.BlockSpec((1, 1), bcast),
  ]
  return pl.pallas_call(
      _tc_body,
      grid=grid,
      in_specs=in_specs,
      out_specs=pl.BlockSpec((BB, 1), lambda i: (i, 0)),
      out_shape=jax.ShapeDtypeStruct((B, 1), jnp.float32),
  )(seq_len2, k_item, k_cate, q_item, q_cate, u_id, u_age, u_gen,
    att_w0, att_b0, att_w1, att_b1, att_w2, att_b2,
    f_w0, f_b0, f_w1, f_b1, f_w2, f_b2, f_w3, f_b3)


def kernel(user_id, user_age, user_gender, item_id, cate_id, hist_item_id,
           hist_cate_id, seq_len, W_user_id, W_user_age, W_user_gender,
           W_item_id, W_cate_id, W_hist_item, W_hist_cate,
           att_W0, att_b0, att_W1, att_b1, att_W2, att_b2,
           f_W0, f_b0, f_W1, f_b1, f_W2, f_b2, f_W3, f_b3):
  mask = jnp.arange(PAD, dtype=jnp.int32)[None, :] < seq_len[:, None]
  pad6 = jnp.zeros((B, PADP - PAD), jnp.int32)
  hi = jnp.concatenate(
      [jnp.where(mask, hist_item_id, 0).astype(jnp.int32), pad6], axis=1)
  hc = jnp.concatenate(
      [jnp.where(mask, hist_cate_id, 0).astype(jnp.int32), pad6], axis=1)
  idx_hist_item = hi.reshape(NW, MH_W)
  idx_hist_cate = hc.reshape(NW, MH_W)

  tables = (_bf16(W_hist_item, 4000), _bf16(W_hist_cate, 1000),
            W_user_id, W_user_age, W_user_gender, W_item_id, W_cate_id)
  idxs = (idx_hist_item, idx_hist_cate, user_id.astype(jnp.int32),
          user_age.astype(jnp.int32), user_gender.astype(jnp.int32),
          item_id.astype(jnp.int32), cate_id.astype(jnp.int32))
  (e_hist_item, e_hist_cate, e_uid, e_uage, e_ugen, e_iid, e_icate) = (
      _sc_gather(tables, idxs))

  seq_len2 = seq_len.astype(jnp.int32).reshape(B, 1)

  r1 = lambda a: a.reshape(1, -1)
  return _tc_dense(
      seq_len2, e_hist_item, e_hist_cate, e_iid, e_icate, e_uid, e_uage, e_ugen,
      att_W0, r1(att_b0), att_W1, r1(att_b1), att_W2, r1(att_b2),
      f_W0, r1(f_b0), f_W1, r1(f_b1), f_W2, r1(f_b2), f_W3, r1(f_b3))


# speedup vs baseline: 1.2134x; 1.2134x over previous
import jax
import jax.numpy as jnp
from jax import lax
from jax.experimental import pallas as pl
from jax.experimental.pallas import tpu as pltpu
from jax.experimental.pallas import tpu_sc as plsc

B = 4096
PAD = 50
EMB = 32
NEG = -1e9

NC = 2
NS = 16
NW = NC * NS

M = B * PAD
MH_W = M // NW
CH = 1600
N_CH = MH_W // CH
SM_W = B // NW


def _sc_gather_body(
    t_hist_item, t_hist_cate, t_uid, t_uage, t_ugen, t_iid, t_icate,
    idx_hist_item, idx_hist_cate, idx_uid, idx_uage, idx_ugen, idx_iid,
    idx_icate,
    o_hist_item, o_hist_cate, o_uid, o_uage, o_ugen, o_iid, o_icate,
    idx2, buf2, idx_sml, buf_sml, ct_sh, gsems, wsems, ssem,
):
  sid = lax.axis_index("s")
  wid = lax.axis_index("c") * NS + sid
  sbase = wid * SM_W

  @pl.when(sid == 0)
  def _():
    pltpu.sync_copy(t_hist_cate, ct_sh)

  pltpu.sync_copy(idx_hist_item.at[wid], idx2.at[0])
  pltpu.sync_copy(idx_hist_cate.at[wid], idx2.at[1])
  plsc.subcore_barrier()

  smalls = ((t_uid, idx_uid, o_uid), (t_uage, idx_uage, o_uage),
            (t_ugen, idx_ugen, o_ugen), (t_iid, idx_iid, o_iid),
            (t_icate, idx_icate, o_icate))
  sm_g = []
  for i, (table, idx_hbm, _) in enumerate(smalls):
    pltpu.sync_copy(idx_hbm.at[pl.ds(sbase, SM_W)], idx_sml.at[i])
    sm_g.append(pltpu.async_copy(table.at[idx_sml.at[i]], buf_sml.at[i], ssem))

  tables = (t_hist_item, ct_sh)
  outs = (o_hist_item, o_hist_cate)
  units = [(t, ch) for t in range(2) for ch in range(N_CH)]
  fires = [None] * len(units)
  wbs = [None] * len(units)

  def fire(u):
    t, ch = units[u]
    b = u % 2
    fires[u] = pltpu.async_copy(
        tables[t].at[idx2.at[t].at[pl.ds(ch * CH, CH)]],
        buf2.at[b], gsems[b])

  def drain_and_writeback(u):
    t, ch = units[u]
    b = u % 2
    fires[u].wait()
    wbs[u] = pltpu.async_copy(
        buf2.at[b], outs[t].at[pl.ds(wid * MH_W + ch * CH, CH)], wsems[b])

  fire(0)
  for u in range(1, len(units)):
    if u >= 2:
      wbs[u - 2].wait()
    fire(u)
    drain_and_writeback(u - 1)
  drain_and_writeback(len(units) - 1)
  wbs[-2].wait()
  wbs[-1].wait()

  for c in sm_g:
    c.wait()
  for i, (_, _, out_hbm) in enumerate(smalls):
    pltpu.sync_copy(buf_sml.at[i], out_hbm.at[pl.ds(sbase, SM_W)])


def _sc_gather(tables, idxs):
  f32 = jnp.float32
  bf16 = jnp.bfloat16
  out_type = [
      jax.ShapeDtypeStruct((M, EMB), bf16),
      jax.ShapeDtypeStruct((M, EMB), bf16),
      jax.ShapeDtypeStruct((B, EMB), f32),
      jax.ShapeDtypeStruct((B, EMB), f32),
      jax.ShapeDtypeStruct((B, EMB), f32),
      jax.ShapeDtypeStruct((B, EMB), f32),
      jax.ShapeDtypeStruct((B, EMB), f32),
  ]
  mesh = plsc.VectorSubcoreMesh(core_axis_name="c", subcore_axis_name="s",
                                num_cores=NC, num_subcores=NS)
  run = pl.kernel(
      _sc_gather_body,
      out_type=out_type,
      mesh=mesh,
      compiler_params=pltpu.CompilerParams(use_tc_tiling_on_sc=False,
                                           disable_bounds_checks=True),
      scratch_types=[
          pltpu.VMEM((2, MH_W), jnp.int32),
          pltpu.VMEM((2, CH, EMB), bf16),
          pltpu.VMEM((5, SM_W), jnp.int32),
          pltpu.VMEM((5, SM_W, EMB), f32),
          pltpu.VMEM_SHARED((1000, EMB), bf16),
          (pltpu.SemaphoreType.DMA, pltpu.SemaphoreType.DMA),
          (pltpu.SemaphoreType.DMA, pltpu.SemaphoreType.DMA),
          pltpu.SemaphoreType.DMA,
      ],
  )
  return run(*tables, *idxs)


BB = 128
MB = BB * PAD


def _tc_body(seq_ref, ki_ref, kc_ref, qi_ref, qc_ref, uid_ref, uage_ref,
             ugen_ref, w0_ref, b0_ref, w1_ref, b1_ref, w2_ref, b2_ref,
             fw0_ref, fb0_ref, fw1_ref, fb1_ref, fw2_ref, fb2_ref,
             fw3_ref, fb3_ref, out_ref):
  f32 = jnp.float32
  qi = qi_ref[...]
  qc = qc_ref[...]
  q = jnp.concatenate([qi, qc], axis=-1)
  w0 = w0_ref[...]
  w0q, w0k, w0d, w0p = (w0[0:64], w0[64:128], w0[128:192], w0[192:256])
  a_row = jnp.dot(q, w0q + w0d, preferred_element_type=f32)

  k3 = jnp.concatenate([ki_ref[...].astype(f32), kc_ref[...].astype(f32)],
                       axis=-1)
  qk3 = q[None, :, :] * k3
  k2 = k3.reshape(MB, 2 * EMB)
  x = jnp.concatenate([k2, qk3.reshape(MB, 2 * EMB)], axis=-1)
  wx = jnp.concatenate([w0k - w0d, w0p], axis=0)
  h = jnp.dot(x, wx, preferred_element_type=f32)
  h += jnp.broadcast_to(a_row[None], (PAD, BB, 80)).reshape(MB, 80)
  h = jnp.maximum(h + b0_ref[...], 0.0)
  h = jnp.maximum(jnp.dot(h, w1_ref[...], preferred_element_type=f32)
                  + b1_ref[...], 0.0)
  s = jnp.dot(h, w2_ref[...], preferred_element_type=f32) + b2_ref[...]
  s3 = s.reshape(PAD, BB, 1)
  pos = lax.broadcasted_iota(jnp.int32, (PAD, BB, 1), 0)
  mask = pos < seq_ref[...].reshape(1, BB, 1)
  s3 = jnp.where(mask, s3, NEG)
  mx = jnp.max(s3, axis=0, keepdims=True)
  e = jnp.exp(s3 - mx)
  att = e / jnp.sum(e, axis=0, keepdims=True)
  seq_emb = jnp.sum(att * k3, axis=0)

  dnn_in = jnp.concatenate(
      [uid_ref[...], uage_ref[...], ugen_ref[...], seq_emb, qi, qc], axis=-1)
  h = jnp.maximum(jnp.dot(dnn_in, fw0_ref[...], preferred_element_type=f32)
                  + fb0_ref[...], 0.0)
  h = jnp.maximum(jnp.dot(h, fw1_ref[...], preferred_element_type=f32)
                  + fb1_ref[...], 0.0)
  h = jnp.maximum(jnp.dot(h, fw2_ref[...], preferred_element_type=f32)
                  + fb2_ref[...], 0.0)
  z = jnp.dot(h, fw3_ref[...], preferred_element_type=f32) + fb3_ref[...]
  out_ref[...] = 1.0 / (1.0 + jnp.exp(-z))


def _tc_dense(seq_len2, k_item, k_cate, q_item, q_cate, u_id, u_age, u_gen,
              att_w0, att_b0, att_w1, att_b1, att_w2, att_b2,
              f_w0, f_b0, f_w1, f_b1, f_w2, f_b2, f_w3, f_b3):
  grid = (B // BB,)
  bcast = lambda i: (0, 0)

  def row_spec(cols):
    return pl.BlockSpec((BB, cols), lambda i: (i, 0))

  in_specs = [
      pl.BlockSpec((BB, 1), lambda i: (i, 0)),
      pl.BlockSpec((PAD, BB, EMB), lambda i: (0, i, 0)),
      pl.BlockSpec((PAD, BB, EMB), lambda i: (0, i, 0)),
      row_spec(EMB), row_spec(EMB),
      row_spec(EMB), row_spec(EMB), row_spec(EMB),
      pl.BlockSpec((256, 80), bcast), pl.BlockSpec((1, 80), bcast),
      pl.BlockSpec((80, 40), bcast), pl.BlockSpec((1, 40), bcast),
      pl.BlockSpec((40, 1), bcast), pl.BlockSpec((1, 1), bcast),
      pl.BlockSpec((224, 256), bcast), pl.BlockSpec((1, 256), bcast),
      pl.BlockSpec((256, 128), bcast), pl.BlockSpec((1, 128), bcast),
      pl.BlockSpec((128, 64), bcast), pl.BlockSpec((1, 64), bcast),
      pl.BlockSpec((64, 1), bcast), pl.BlockSpec((1, 1), bcast),
  ]
  return pl.pallas_call(
      _tc_body,
      grid=grid,
      in_specs=in_specs,
      out_specs=pl.BlockSpec((BB, 1), lambda i: (i, 0)),
      out_shape=jax.ShapeDtypeStruct((B, 1), jnp.float32),
  )(seq_len2, k_item, k_cate, q_item, q_cate, u_id, u_age, u_gen,
    att_w0, att_b0, att_w1, att_b1, att_w2, att_b2,
    f_w0, f_b0, f_w1, f_b1, f_w2, f_b2, f_w3, f_b3)


def kernel(user_id, user_age, user_gender, item_id, cate_id, hist_item_id,
           hist_cate_id, seq_len, W_user_id, W_user_age, W_user_gender,
           W_item_id, W_cate_id, W_hist_item, W_hist_cate,
           att_W0, att_b0, att_W1, att_b1, att_W2, att_b2,
           f_W0, f_b0, f_W1, f_b1, f_W2, f_b2, f_W3, f_b3):
  mask = jnp.arange(PAD, dtype=jnp.int32)[None, :] < seq_len[:, None]
  hi = jnp.where(mask, hist_item_id, 0).astype(jnp.int32)
  hc = jnp.where(mask, hist_cate_id, 0).astype(jnp.int32)
  idx_hist_item = hi.T.reshape(NW, MH_W)
  idx_hist_cate = hc.T.reshape(NW, MH_W)

  tables = (W_hist_item.astype(jnp.bfloat16), W_hist_cate.astype(jnp.bfloat16),
            W_user_id, W_user_age, W_user_gender, W_item_id, W_cate_id)
  idxs = (idx_hist_item, idx_hist_cate, user_id.astype(jnp.int32),
          user_age.astype(jnp.int32), user_gender.astype(jnp.int32),
          item_id.astype(jnp.int32), cate_id.astype(jnp.int32))
  (e_hist_item, e_hist_cate, e_uid, e_uage, e_ugen, e_iid, e_icate) = (
      _sc_gather(tables, idxs))

  k_item = e_hist_item.reshape(PAD, B, EMB)
  k_cate = e_hist_cate.reshape(PAD, B, EMB)
  seq_len2 = seq_len.astype(jnp.int32).reshape(B, 1)

  r1 = lambda a: a.reshape(1, -1)
  return _tc_dense(
      seq_len2, k_item, k_cate, e_iid, e_icate, e_uid, e_uage, e_ugen,
      att_W0, r1(att_b0), att_W1, r1(att_b1), att_W2, r1(att_b2),
      f_W0, r1(f_b0), f_W1, r1(f_b1), f_W2, r1(f_b2), f_W3, r1(f_b3))
